# Initial kernel scaffold; baseline (speedup 1.0000x reference)
import jax
import jax.numpy as jnp
from jax.experimental import pallas as pl


def kernel(features, edge_index, W1, b1, W2, b2, Wg, bg, attn_l, attn_r, W3, b3):
    raise NotImplementedError("write your pallas kernel here")


# speedup vs baseline: 1.0001x; 1.0001x over previous
import jax
import jax.numpy as jnp
from jax.experimental import pallas as pl

N = 10000
E = 320000
D = 128
H = 64
HEADS = 4


def _gc(x, src, dst, W, b, n):
    deg_out = jnp.clip(jnp.bincount(src, length=n), 1).astype(x.dtype)
    deg_in = jnp.clip(jnp.bincount(dst, length=n), 1).astype(x.dtype)
    norm_src = 1.0 / jnp.sqrt(deg_out)
    norm_dst = 1.0 / jnp.sqrt(deg_in)
    h = x @ W
    m = h[src] * norm_src[src][:, None]
    agg = jax.ops.segment_sum(m, dst, num_segments=n)
    return agg * norm_dst[:, None] + b


def _gat(x, src, dst, Wg, bg, attn_l, attn_r, n):
    feat = (x @ Wg).reshape(n, HEADS, H)
    el = (feat * attn_l[None, :, :]).sum(-1)
    er = (feat * attn_r[None, :, :]).sum(-1)
    e = jax.nn.leaky_relu(el[src] + er[dst], negative_slope=0.2)
    emax = jax.ops.segment_max(e, dst, num_segments=n)
    eexp = jnp.exp(e - emax[dst])
    denom = jax.ops.segment_sum(eexp, dst, num_segments=n)
    alpha = eexp / denom[dst]
    msg = feat[src] * alpha[:, :, None]
    out = jax.ops.segment_sum(msg, dst, num_segments=n)
    out = out + bg.reshape(1, HEADS, H)
    return out, alpha


def kernel(features, edge_index, W1, b1, W2, b2, Wg, bg, attn_l, attn_r, W3, b3):
    src = edge_index[0]
    dst = edge_index[1]
    h = jax.nn.relu(_gc(features, src, dst, W1, b1, N))
    h = jax.nn.relu(_gc(h, src, dst, W2, b2, N))
    h, attention_weights = _gat(h, src, dst, Wg, bg, attn_l, attn_r, N)
    h = h.mean(axis=1)
    risk_scores = jax.nn.sigmoid(_gc(h, src, dst, W3, b3, N))
    top_vals, top_idx = jax.lax.top_k(risk_scores.squeeze(-1), 5)
    risk_score = risk_scores.mean()
    confidence = attention_weights.mean()
    return risk_scores, risk_score, confidence, top_idx


# speedup vs baseline: 7.9823x; 7.9817x over previous
import functools

import jax
import jax.numpy as jnp
from jax import lax
from jax.experimental import pallas as pl
from jax.experimental.pallas import tpu as pltpu
from jax.experimental.pallas import tpu_sc as plsc

N = 10000
E = 320000
D = 128
H = 64
HEADS = 4

_NC = 2
_NS = 16
_C = 128
_E_SC = E // _NC
_CHUNKS = _E_SC // _C
_CH_PER_TILE = -(-_CHUNKS // _NS)
_ZROWS = 200
_NBLK = N // _ZROWS
_BLK_PER_TILE = -(-_NBLK // _NS)


def _zero_fill2d(buf, rows, width):
    z = jnp.zeros((16,), jnp.float32)
    per_row = width // 16

    def body(i, _):
        buf[i // per_row, pl.ds(pl.multiple_of((i % per_row) * 16, 16), 16)] = z
        return 0

    lax.fori_loop(0, rows * per_row, body, 0)


def _seg_sum_rows(table, src, dst, width):
    mesh = plsc.VectorSubcoreMesh(core_axis_name="c", subcore_axis_name="s")

    @functools.partial(
        pl.kernel,
        out_type=jax.ShapeDtypeStruct((_NC, N, width), jnp.float32),
        mesh=mesh,
        scratch_types=[
            pltpu.VMEM((_C,), jnp.int32),
            pltpu.VMEM((_C,), jnp.int32),
            pltpu.VMEM((_C, width), jnp.float32),
            pltpu.VMEM((_ZROWS, width), jnp.float32),
            pltpu.VMEM_SHARED((N, width), jnp.float32),
            pltpu.SemaphoreType.DMA,
        ],
        compiler_params=pltpu.CompilerParams(use_tc_tiling_on_sc=False),
    )
    def k(table_hbm, src_hbm, dst_hbm, out_hbm, idx_s, idx_d, rows, zbuf,
          acc_sh, sem):
        c = lax.axis_index("c")
        s = lax.axis_index("s")

        _zero_fill2d(zbuf, _ZROWS, width)

        def zs(j, _):
            b = s + j * _NS

            @pl.when(b < _NBLK)
            def _():
                pltpu.sync_copy(zbuf, acc_sh.at[pl.ds(pl.multiple_of(b * _ZROWS, _ZROWS), _ZROWS)])

            return 0

        lax.fori_loop(0, _BLK_PER_TILE, zs, 0)
        plsc.subcore_barrier()

        def chunk(i, _):
            kk = s + i * _NS

            @pl.when(kk < _CHUNKS)
            def _():
                base = c * _E_SC + kk * _C
                pltpu.sync_copy(src_hbm.at[pl.ds(base, _C)], idx_s)
                pltpu.sync_copy(dst_hbm.at[pl.ds(base, _C)], idx_d)
                pltpu.async_copy(table_hbm.at[idx_s], rows, sem).wait()
                pltpu.sync_copy(rows, acc_sh.at[idx_d], add=True)

            return 0

        lax.fori_loop(0, _CH_PER_TILE, chunk, 0)
        plsc.subcore_barrier()

        def wr(j, _):
            b = s + j * _NS

            @pl.when(b < _NBLK)
            def _():
                off = pl.multiple_of(b * _ZROWS, _ZROWS)
                pltpu.sync_copy(acc_sh.at[pl.ds(off, _ZROWS)],
                                out_hbm.at[c, pl.ds(off, _ZROWS)])

            return 0

        lax.fori_loop(0, _BLK_PER_TILE, wr, 0)

    return k(table, src, dst)


_DW = 16


def _degrees(src, dst):
    mesh = plsc.VectorSubcoreMesh(core_axis_name="c", subcore_axis_name="s")

    @functools.partial(
        pl.kernel,
        out_type=(jax.ShapeDtypeStruct((_NC, N, _DW), jnp.float32),
                  jax.ShapeDtypeStruct((_NC, N, _DW), jnp.float32)),
        mesh=mesh,
        scratch_types=[
            pltpu.VMEM((_C,), jnp.int32),
            pltpu.VMEM((_C,), jnp.int32),
            pltpu.VMEM((_C, _DW), jnp.float32),
            pltpu.VMEM((_ZROWS, _DW), jnp.float32),
            pltpu.VMEM_SHARED((N, _DW), jnp.float32),
            pltpu.VMEM_SHARED((N, _DW), jnp.float32),
        ],
        compiler_params=pltpu.CompilerParams(use_tc_tiling_on_sc=False),
    )
    def k(src_hbm, dst_hbm, osrc_hbm, odst_hbm, idx_s, idx_d, ones, zbuf,
          acc_s, acc_d):
        c = lax.axis_index("c")
        s = lax.axis_index("s")

        _zero_fill2d(zbuf, _ZROWS, _DW)
        o = jnp.ones((16,), jnp.float32)

        def fill(i, _):
            ones[i, pl.ds(0, 16)] = o
            return 0

        lax.fori_loop(0, _C, fill, 0)

        def zs(j, _):
            b = s + j * _NS

            @pl.when(b < _NBLK)
            def _():
                off = pl.multiple_of(b * _ZROWS, _ZROWS)
                pltpu.sync_copy(zbuf, acc_s.at[pl.ds(off, _ZROWS)])
                pltpu.sync_copy(zbuf, acc_d.at[pl.ds(off, _ZROWS)])

            return 0

        lax.fori_loop(0, _BLK_PER_TILE, zs, 0)
        plsc.subcore_barrier()

        def chunk(i, _):
            kk = s + i * _NS

            @pl.when(kk < _CHUNKS)
            def _():
                base = c * _E_SC + kk * _C
                pltpu.sync_copy(src_hbm.at[pl.ds(base, _C)], idx_s)
                pltpu.sync_copy(dst_hbm.at[pl.ds(base, _C)], idx_d)
                pltpu.sync_copy(ones, acc_s.at[idx_s], add=True)
                pltpu.sync_copy(ones, acc_d.at[idx_d], add=True)

            return 0

        lax.fori_loop(0, _CH_PER_TILE, chunk, 0)
        plsc.subcore_barrier()

        def wr(j, _):
            b = s + j * _NS

            @pl.when(b < _NBLK)
            def _():
                off = pl.multiple_of(b * _ZROWS, _ZROWS)
                pltpu.sync_copy(acc_s.at[pl.ds(off, _ZROWS)],
                                osrc_hbm.at[c, pl.ds(off, _ZROWS)])
                pltpu.sync_copy(acc_d.at[pl.ds(off, _ZROWS)],
                                odst_hbm.at[c, pl.ds(off, _ZROWS)])

            return 0

        lax.fori_loop(0, _BLK_PER_TILE, wr, 0)

    return k(src, dst)


def _gc(x, src, dst, W, b, n, norm_src, norm_dst, width):
    h = x @ W
    if width > h.shape[1]:
        h = jnp.broadcast_to(h, (n, width))
    table = h * norm_src[:, None]
    parts = _seg_sum_rows(table, src, dst, width)
    agg = parts[0] + parts[1]
    return agg * norm_dst[:, None] + b


def _gat_edge_weights(ta, tb, src, dst):
    mesh = plsc.VectorSubcoreMesh(core_axis_name="c", subcore_axis_name="s")

    @functools.partial(
        pl.kernel,
        out_type=(jax.ShapeDtypeStruct((E, 16), jnp.float32),
                  jax.ShapeDtypeStruct((_NC, N, 16), jnp.float32)),
        mesh=mesh,
        scratch_types=[
            pltpu.VMEM((_C,), jnp.int32),
            pltpu.VMEM((_C,), jnp.int32),
            pltpu.VMEM((_C, 16), jnp.float32),
            pltpu.VMEM((_C, 16), jnp.float32),
            pltpu.VMEM((_C, 16), jnp.float32),
            pltpu.VMEM((_ZROWS, 16), jnp.float32),
            pltpu.VMEM_SHARED((N, 16), jnp.float32),
            pltpu.SemaphoreType.DMA,
        ],
        compiler_params=pltpu.CompilerParams(use_tc_tiling_on_sc=False),
    )
    def k(ta_hbm, tb_hbm, src_hbm, dst_hbm, w_hbm, den_hbm, idx_s, idx_d,
          es, ed, wbuf, zbuf, acc_sh, sem):
        c = lax.axis_index("c")
        s = lax.axis_index("s")

        _zero_fill2d(zbuf, _ZROWS, 16)

        def zs(j, _):
            b = s + j * _NS

            @pl.when(b < _NBLK)
            def _():
                off = pl.multiple_of(b * _ZROWS, _ZROWS)
                pltpu.sync_copy(zbuf, acc_sh.at[pl.ds(off, _ZROWS)])

            return 0

        lax.fori_loop(0, _BLK_PER_TILE, zs, 0)
        plsc.subcore_barrier()

        def chunk(i, _):
            kk = s + i * _NS

            @pl.when(kk < _CHUNKS)
            def _():
                base = c * _E_SC + kk * _C
                pltpu.sync_copy(src_hbm.at[pl.ds(base, _C)], idx_s)
                pltpu.sync_copy(dst_hbm.at[pl.ds(base, _C)], idx_d)
                cp1 = pltpu.async_copy(ta_hbm.at[idx_s], es, sem)
                cp2 = pltpu.async_copy(tb_hbm.at[idx_d], ed, sem)
                cp1.wait()
                cp2.wait()

                def pe(j, _):
                    e2 = es[j, pl.ds(0, 16)] + ed[j, pl.ds(0, 16)]
                    e2 = jnp.where(e2 > 0, e2, 0.2 * e2)
                    wbuf[j, pl.ds(0, 16)] = jnp.exp(e2)
                    return 0

                lax.fori_loop(0, _C, pe, 0)
                pltpu.sync_copy(wbuf, w_hbm.at[pl.ds(base, _C)])
                pltpu.sync_copy(wbuf, acc_sh.at[idx_d], add=True)

            return 0

        lax.fori_loop(0, _CH_PER_TILE, chunk, 0)
        plsc.subcore_barrier()

        def wr(j, _):
            b = s + j * _NS

            @pl.when(b < _NBLK)
            def _():
                off = pl.multiple_of(b * _ZROWS, _ZROWS)
                pltpu.sync_copy(acc_sh.at[pl.ds(off, _ZROWS)],
                                den_hbm.at[c, pl.ds(off, _ZROWS)])

            return 0

        lax.fori_loop(0, _BLK_PER_TILE, wr, 0)

    return k(ta, tb, src, dst)


def _gat_aggregate(feat, w, den2, src, dst):
    mesh = plsc.VectorSubcoreMesh(core_axis_name="c", subcore_axis_name="s")

    @functools.partial(
        pl.kernel,
        out_type=jax.ShapeDtypeStruct((_NC, N, H), jnp.float32),
        mesh=mesh,
        scratch_types=[
            pltpu.VMEM((_C,), jnp.int32),
            pltpu.VMEM((_C,), jnp.int32),
            pltpu.VMEM((_C,), jnp.int32),
            pltpu.VMEM((_C, HEADS * H), jnp.float32),
            pltpu.VMEM((_C, 16), jnp.float32),
            pltpu.VMEM((_C, 16), jnp.float32),
            pltpu.VMEM((_C, 16), jnp.float32),
            pltpu.VMEM((_C, H), jnp.float32),
            pltpu.VMEM((_ZROWS, H), jnp.float32),
            pltpu.VMEM_SHARED((N, H), jnp.float32),
            pltpu.SemaphoreType.DMA,
        ],
        compiler_params=pltpu.CompilerParams(use_tc_tiling_on_sc=False),
    )
    def k(feat_hbm, w_hbm, den_hbm, src_hbm, dst_hbm, out_hbm, idx_s, idx_d,
          idx_d2, frows, wch, d0, d1, msg, zbuf, acc_sh, sem):
        c = lax.axis_index("c")
        s = lax.axis_index("s")

        _zero_fill2d(zbuf, _ZROWS, H)

        def zs(j, _):
            b = s + j * _NS

            @pl.when(b < _NBLK)
            def _():
                off = pl.multiple_of(b * _ZROWS, _ZROWS)
                pltpu.sync_copy(zbuf, acc_sh.at[pl.ds(off, _ZROWS)])

            return 0

        lax.fori_loop(0, _BLK_PER_TILE, zs, 0)
        plsc.subcore_barrier()

        bidx = [jnp.full((16,), h, jnp.int32) for h in range(HEADS)]

        def chunk(i, _):
            kk = s + i * _NS

            @pl.when(kk < _CHUNKS)
            def _():
                base = c * _E_SC + kk * _C
                pltpu.sync_copy(src_hbm.at[pl.ds(base, _C)], idx_s)
                pltpu.sync_copy(dst_hbm.at[pl.ds(base, _C)], idx_d)

                def sh(q, _):
                    off = pl.multiple_of(q * 16, 16)
                    idx_d2[pl.ds(off, 16)] = idx_d[pl.ds(off, 16)] + N
                    return 0

                lax.fori_loop(0, _C // 16, sh, 0)

                cp1 = pltpu.async_copy(feat_hbm.at[idx_s], frows, sem)
                pltpu.sync_copy(w_hbm.at[pl.ds(base, _C)], wch)
                cp2 = pltpu.async_copy(den_hbm.at[idx_d], d0, sem)
                cp3 = pltpu.async_copy(den_hbm.at[idx_d2], d1, sem)
                cp1.wait()
                cp2.wait()
                cp3.wait()

                def pe(j, _):
                    dv = d0[j, pl.ds(0, 16)] + d1[j, pl.ds(0, 16)]
                    dv = jnp.where(dv == 0.0, 1.0, dv)
                    cf = 0.25 * wch[j, pl.ds(0, 16)] / dv
                    cb = [cf.at[bidx[h]].get(mode=jax.lax.GatherScatterMode.PROMISE_IN_BOUNDS)
                          for h in range(HEADS)]
                    for q in range(H // 16):
                        o = q * 16
                        a = cb[0] * frows[j, pl.ds(0 * H + o, 16)]
                        a = a + cb[1] * frows[j, pl.ds(1 * H + o, 16)]
                        a = a + cb[2] * frows[j, pl.ds(2 * H + o, 16)]
                        a = a + cb[3] * frows[j, pl.ds(3 * H + o, 16)]
                        msg[j, pl.ds(o, 16)] = a
                    return 0

                lax.fori_loop(0, _C, pe, 0)
                pltpu.sync_copy(msg, acc_sh.at[idx_d], add=True)

            return 0

        lax.fori_loop(0, _CH_PER_TILE, chunk, 0)
        plsc.subcore_barrier()

        def wr(j, _):
            b = s + j * _NS

            @pl.when(b < _NBLK)
            def _():
                off = pl.multiple_of(b * _ZROWS, _ZROWS)
                pltpu.sync_copy(acc_sh.at[pl.ds(off, _ZROWS)],
                                out_hbm.at[c, pl.ds(off, _ZROWS)])

            return 0

        lax.fori_loop(0, _BLK_PER_TILE, wr, 0)

    return k(feat, w, den2, src, dst)


def _gat(x, src, dst, Wg, bg, attn_l, attn_r, n):
    feat = (x @ Wg).reshape(n, HEADS, H)
    el = (feat * attn_l[None, :, :]).sum(-1)
    er = (feat * attn_r[None, :, :]).sum(-1)
    e = jax.nn.leaky_relu(el[src] + er[dst], negative_slope=0.2)
    emax = jax.ops.segment_max(e, dst, num_segments=n)
    eexp = jnp.exp(e - emax[dst])
    denom = jax.ops.segment_sum(eexp, dst, num_segments=n)
    alpha = eexp / denom[dst]
    msg = feat[src] * alpha[:, :, None]
    out = jax.ops.segment_sum(msg, dst, num_segments=n)
    out = out + bg.reshape(1, HEADS, H)
    return out, alpha


def kernel(features, edge_index, W1, b1, W2, b2, Wg, bg, attn_l, attn_r, W3, b3):
    src = edge_index[0]
    dst = edge_index[1]
    dsrc_p, ddst_p = _degrees(src, dst)
    deg_out = dsrc_p[0, :, 0] + dsrc_p[1, :, 0]
    deg_in_raw = ddst_p[0, :, 0] + ddst_p[1, :, 0]
    norm_src = 1.0 / jnp.sqrt(jnp.maximum(deg_out, 1.0))
    norm_dst = 1.0 / jnp.sqrt(jnp.maximum(deg_in_raw, 1.0))

    h = jax.nn.relu(_gc(features, src, dst, W1, b1, N, norm_src, norm_dst, H))
    h = jax.nn.relu(_gc(h, src, dst, W2, b2, N, norm_src, norm_dst, H))
    feat2d = h @ Wg
    feat = feat2d.reshape(N, HEADS, H)
    el = (feat * attn_l[None, :, :]).sum(-1)
    er = (feat * attn_r[None, :, :]).sum(-1)
    pad = jnp.zeros((N, 16 - HEADS), jnp.float32)
    ta = jnp.concatenate([el, pad], axis=1)
    tb = jnp.concatenate([er, pad], axis=1)
    w, den_p = _gat_edge_weights(ta, tb, src, dst)
    out_p = _gat_aggregate(feat2d, w, den_p.reshape(2 * N, 16), src, dst)
    h = out_p[0] + out_p[1] + bg.reshape(HEADS, H).mean(0)[None, :]
    hh3 = h @ W3
    m3 = hh3[src] * norm_src[src][:, None]
    agg3 = jax.ops.segment_sum(m3, dst, num_segments=N)
    risk_scores = jax.nn.sigmoid(agg3 * norm_dst[:, None] + b3)
    top_vals, top_idx = jax.lax.top_k(risk_scores.squeeze(-1), 5)
    risk_score = risk_scores.mean()
    confidence = ((ddst_p[0, :, 0] + ddst_p[1, :, 0]) > 0).sum().astype(jnp.float32) / float(E)
    return risk_scores, risk_score, confidence, top_idx


# speedup vs baseline: 26.3015x; 3.2950x over previous
import functools

import jax
import jax.numpy as jnp
from jax import lax
from jax.experimental import pallas as pl
from jax.experimental.pallas import tpu as pltpu
from jax.experimental.pallas import tpu_sc as plsc

N = 10000
E = 320000
D = 128
H = 64
HEADS = 4

_NC = 2
_NS = 16
_C = 128
_E_SC = E // _NC
_CHUNKS = _E_SC // _C
_CH_PER_TILE = -(-_CHUNKS // _NS)
_ZROWS = 200
_NBLK = N // _ZROWS
_BLK_PER_TILE = -(-_NBLK // _NS)


def _zero_fill2d(buf, rows, width):
    z = jnp.zeros((16,), jnp.float32)
    per_row = width // 16

    def body(i, _):
        buf[i // per_row, pl.ds(pl.multiple_of((i % per_row) * 16, 16), 16)] = z
        return 0

    lax.fori_loop(0, rows * per_row, body, 0)


def _seg_sum_rows(table, src, dst, width):
    mesh = plsc.VectorSubcoreMesh(core_axis_name="c", subcore_axis_name="s")

    @functools.partial(
        pl.kernel,
        out_type=jax.ShapeDtypeStruct((_NC, N, width), jnp.float32),
        mesh=mesh,
        scratch_types=[
            pltpu.VMEM((_C,), jnp.int32),
            pltpu.VMEM((_C,), jnp.int32),
            pltpu.VMEM((_C, width), jnp.float32),
            pltpu.VMEM((_ZROWS, width), jnp.float32),
            pltpu.VMEM_SHARED((N, width), jnp.float32),
            pltpu.SemaphoreType.DMA,
        ],
        compiler_params=pltpu.CompilerParams(use_tc_tiling_on_sc=False),
    )
    def k(table_hbm, src_hbm, dst_hbm, out_hbm, idx_s, idx_d, rows, zbuf,
          acc_sh, sem):
        c = lax.axis_index("c")
        s = lax.axis_index("s")

        _zero_fill2d(zbuf, _ZROWS, width)

        def zs(j, _):
            b = s + j * _NS

            @pl.when(b < _NBLK)
            def _():
                pltpu.sync_copy(zbuf, acc_sh.at[pl.ds(pl.multiple_of(b * _ZROWS, _ZROWS), _ZROWS)])

            return 0

        lax.fori_loop(0, _BLK_PER_TILE, zs, 0)
        plsc.subcore_barrier()

        def chunk(i, _):
            kk = s + i * _NS

            @pl.when(kk < _CHUNKS)
            def _():
                base = c * _E_SC + kk * _C
                pltpu.sync_copy(src_hbm.at[pl.ds(base, _C)], idx_s)
                pltpu.sync_copy(dst_hbm.at[pl.ds(base, _C)], idx_d)
                pltpu.async_copy(table_hbm.at[idx_s], rows, sem).wait()
                pltpu.sync_copy(rows, acc_sh.at[idx_d], add=True)

            return 0

        lax.fori_loop(0, _CH_PER_TILE, chunk, 0)
        plsc.subcore_barrier()

        def wr(j, _):
            b = s + j * _NS

            @pl.when(b < _NBLK)
            def _():
                off = pl.multiple_of(b * _ZROWS, _ZROWS)
                pltpu.sync_copy(acc_sh.at[pl.ds(off, _ZROWS)],
                                out_hbm.at[c, pl.ds(off, _ZROWS)])

            return 0

        lax.fori_loop(0, _BLK_PER_TILE, wr, 0)

    return k(table, src, dst)


_DW = 16


def _degrees(src, dst):
    mesh = plsc.VectorSubcoreMesh(core_axis_name="c", subcore_axis_name="s")

    @functools.partial(
        pl.kernel,
        out_type=(jax.ShapeDtypeStruct((_NC, N, _DW), jnp.float32),
                  jax.ShapeDtypeStruct((_NC, N, _DW), jnp.float32)),
        mesh=mesh,
        scratch_types=[
            pltpu.VMEM((_C,), jnp.int32),
            pltpu.VMEM((_C,), jnp.int32),
            pltpu.VMEM((_C, _DW), jnp.float32),
            pltpu.VMEM((_ZROWS, _DW), jnp.float32),
            pltpu.VMEM_SHARED((N, _DW), jnp.float32),
            pltpu.VMEM_SHARED((N, _DW), jnp.float32),
        ],
        compiler_params=pltpu.CompilerParams(use_tc_tiling_on_sc=False),
    )
    def k(src_hbm, dst_hbm, osrc_hbm, odst_hbm, idx_s, idx_d, ones, zbuf,
          acc_s, acc_d):
        c = lax.axis_index("c")
        s = lax.axis_index("s")

        _zero_fill2d(zbuf, _ZROWS, _DW)
        o = jnp.ones((16,), jnp.float32)

        def fill(i, _):
            ones[i, pl.ds(0, 16)] = o
            return 0

        lax.fori_loop(0, _C, fill, 0)

        def zs(j, _):
            b = s + j * _NS

            @pl.when(b < _NBLK)
            def _():
                off = pl.multiple_of(b * _ZROWS, _ZROWS)
                pltpu.sync_copy(zbuf, acc_s.at[pl.ds(off, _ZROWS)])
                pltpu.sync_copy(zbuf, acc_d.at[pl.ds(off, _ZROWS)])

            return 0

        lax.fori_loop(0, _BLK_PER_TILE, zs, 0)
        plsc.subcore_barrier()

        def chunk(i, _):
            kk = s + i * _NS

            @pl.when(kk < _CHUNKS)
            def _():
                base = c * _E_SC + kk * _C
                pltpu.sync_copy(src_hbm.at[pl.ds(base, _C)], idx_s)
                pltpu.sync_copy(dst_hbm.at[pl.ds(base, _C)], idx_d)
                pltpu.sync_copy(ones, acc_s.at[idx_s], add=True)
                pltpu.sync_copy(ones, acc_d.at[idx_d], add=True)

            return 0

        lax.fori_loop(0, _CH_PER_TILE, chunk, 0)
        plsc.subcore_barrier()

        def wr(j, _):
            b = s + j * _NS

            @pl.when(b < _NBLK)
            def _():
                off = pl.multiple_of(b * _ZROWS, _ZROWS)
                pltpu.sync_copy(acc_s.at[pl.ds(off, _ZROWS)],
                                osrc_hbm.at[c, pl.ds(off, _ZROWS)])
                pltpu.sync_copy(acc_d.at[pl.ds(off, _ZROWS)],
                                odst_hbm.at[c, pl.ds(off, _ZROWS)])

            return 0

        lax.fori_loop(0, _BLK_PER_TILE, wr, 0)

    return k(src, dst)


def _gc(x, src, dst, W, b, n, norm_src, norm_dst, width):
    h = x @ W
    if width > h.shape[1]:
        h = jnp.broadcast_to(h, (n, width))
    table = h * norm_src[:, None]
    parts = _seg_sum_rows(table, src, dst, width)
    agg = parts[0] + parts[1]
    return agg * norm_dst[:, None] + b


def _gat_edge_weights(ta, tb, src, dst):
    mesh = plsc.VectorSubcoreMesh(core_axis_name="c", subcore_axis_name="s")

    @functools.partial(
        pl.kernel,
        out_type=(jax.ShapeDtypeStruct((E, 16), jnp.float32),
                  jax.ShapeDtypeStruct((_NC, N, 16), jnp.float32)),
        mesh=mesh,
        scratch_types=[
            pltpu.VMEM((_C,), jnp.int32),
            pltpu.VMEM((_C,), jnp.int32),
            pltpu.VMEM((_C, 16), jnp.float32),
            pltpu.VMEM((_C, 16), jnp.float32),
            pltpu.VMEM((_C, 16), jnp.float32),
            pltpu.VMEM((_ZROWS, 16), jnp.float32),
            pltpu.VMEM_SHARED((N, 16), jnp.float32),
            pltpu.SemaphoreType.DMA,
        ],
        compiler_params=pltpu.CompilerParams(use_tc_tiling_on_sc=False),
    )
    def k(ta_hbm, tb_hbm, src_hbm, dst_hbm, w_hbm, den_hbm, idx_s, idx_d,
          es, ed, wbuf, zbuf, acc_sh, sem):
        c = lax.axis_index("c")
        s = lax.axis_index("s")

        _zero_fill2d(zbuf, _ZROWS, 16)

        def zs(j, _):
            b = s + j * _NS

            @pl.when(b < _NBLK)
            def _():
                off = pl.multiple_of(b * _ZROWS, _ZROWS)
                pltpu.sync_copy(zbuf, acc_sh.at[pl.ds(off, _ZROWS)])

            return 0

        lax.fori_loop(0, _BLK_PER_TILE, zs, 0)
        plsc.subcore_barrier()

        def chunk(i, _):
            kk = s + i * _NS

            @pl.when(kk < _CHUNKS)
            def _():
                base = c * _E_SC + kk * _C
                pltpu.sync_copy(src_hbm.at[pl.ds(base, _C)], idx_s)
                pltpu.sync_copy(dst_hbm.at[pl.ds(base, _C)], idx_d)
                cp1 = pltpu.async_copy(ta_hbm.at[idx_s], es, sem)
                cp2 = pltpu.async_copy(tb_hbm.at[idx_d], ed, sem)
                cp1.wait()
                cp2.wait()

                def pe(j, _):
                    e2 = es[j, pl.ds(0, 16)] + ed[j, pl.ds(0, 16)]
                    e2 = jnp.where(e2 > 0, e2, 0.2 * e2)
                    wbuf[j, pl.ds(0, 16)] = jnp.exp(e2)
                    return 0

                lax.fori_loop(0, _C, pe, 0)
                pltpu.sync_copy(wbuf, w_hbm.at[pl.ds(base, _C)])
                pltpu.sync_copy(wbuf, acc_sh.at[idx_d], add=True)

            return 0

        lax.fori_loop(0, _CH_PER_TILE, chunk, 0)
        plsc.subcore_barrier()

        def wr(j, _):
            b = s + j * _NS

            @pl.when(b < _NBLK)
            def _():
                off = pl.multiple_of(b * _ZROWS, _ZROWS)
                pltpu.sync_copy(acc_sh.at[pl.ds(off, _ZROWS)],
                                den_hbm.at[c, pl.ds(off, _ZROWS)])

            return 0

        lax.fori_loop(0, _BLK_PER_TILE, wr, 0)

    return k(ta, tb, src, dst)


def _gat_aggregate(feat, w, den2, src, dst):
    mesh = plsc.VectorSubcoreMesh(core_axis_name="c", subcore_axis_name="s")

    @functools.partial(
        pl.kernel,
        out_type=jax.ShapeDtypeStruct((_NC, N, H), jnp.float32),
        mesh=mesh,
        scratch_types=[
            pltpu.VMEM((_C,), jnp.int32),
            pltpu.VMEM((_C,), jnp.int32),
            pltpu.VMEM((_C,), jnp.int32),
            pltpu.VMEM((_C, HEADS * H), jnp.float32),
            pltpu.VMEM((_C, 16), jnp.float32),
            pltpu.VMEM((_C, 16), jnp.float32),
            pltpu.VMEM((_C, 16), jnp.float32),
            pltpu.VMEM((_C, H), jnp.float32),
            pltpu.VMEM((_ZROWS, H), jnp.float32),
            pltpu.VMEM_SHARED((N, H), jnp.float32),
            pltpu.SemaphoreType.DMA,
        ],
        compiler_params=pltpu.CompilerParams(use_tc_tiling_on_sc=False),
    )
    def k(feat_hbm, w_hbm, den_hbm, src_hbm, dst_hbm, out_hbm, idx_s, idx_d,
          idx_d2, frows, wch, d0, d1, msg, zbuf, acc_sh, sem):
        c = lax.axis_index("c")
        s = lax.axis_index("s")

        _zero_fill2d(zbuf, _ZROWS, H)

        def zs(j, _):
            b = s + j * _NS

            @pl.when(b < _NBLK)
            def _():
                off = pl.multiple_of(b * _ZROWS, _ZROWS)
                pltpu.sync_copy(zbuf, acc_sh.at[pl.ds(off, _ZROWS)])

            return 0

        lax.fori_loop(0, _BLK_PER_TILE, zs, 0)
        plsc.subcore_barrier()

        bidx = [jnp.full((16,), h, jnp.int32) for h in range(HEADS)]

        def chunk(i, _):
            kk = s + i * _NS

            @pl.when(kk < _CHUNKS)
            def _():
                base = c * _E_SC + kk * _C
                pltpu.sync_copy(src_hbm.at[pl.ds(base, _C)], idx_s)
                pltpu.sync_copy(dst_hbm.at[pl.ds(base, _C)], idx_d)

                def sh(q, _):
                    off = pl.multiple_of(q * 16, 16)
                    idx_d2[pl.ds(off, 16)] = idx_d[pl.ds(off, 16)] + N
                    return 0

                lax.fori_loop(0, _C // 16, sh, 0)

                cp1 = pltpu.async_copy(feat_hbm.at[idx_s], frows, sem)
                pltpu.sync_copy(w_hbm.at[pl.ds(base, _C)], wch)
                cp2 = pltpu.async_copy(den_hbm.at[idx_d], d0, sem)
                cp3 = pltpu.async_copy(den_hbm.at[idx_d2], d1, sem)
                cp1.wait()
                cp2.wait()
                cp3.wait()

                def pe(j, _):
                    dv = d0[j, pl.ds(0, 16)] + d1[j, pl.ds(0, 16)]
                    dv = jnp.where(dv == 0.0, 1.0, dv)
                    cf = 0.25 * wch[j, pl.ds(0, 16)] / dv
                    cb = [cf.at[bidx[h]].get(mode=jax.lax.GatherScatterMode.PROMISE_IN_BOUNDS)
                          for h in range(HEADS)]
                    for q in range(H // 16):
                        o = q * 16
                        a = cb[0] * frows[j, pl.ds(0 * H + o, 16)]
                        a = a + cb[1] * frows[j, pl.ds(1 * H + o, 16)]
                        a = a + cb[2] * frows[j, pl.ds(2 * H + o, 16)]
                        a = a + cb[3] * frows[j, pl.ds(3 * H + o, 16)]
                        msg[j, pl.ds(o, 16)] = a
                    return 0

                lax.fori_loop(0, _C, pe, 0)
                pltpu.sync_copy(msg, acc_sh.at[idx_d], add=True)

            return 0

        lax.fori_loop(0, _CH_PER_TILE, chunk, 0)
        plsc.subcore_barrier()

        def wr(j, _):
            b = s + j * _NS

            @pl.when(b < _NBLK)
            def _():
                off = pl.multiple_of(b * _ZROWS, _ZROWS)
                pltpu.sync_copy(acc_sh.at[pl.ds(off, _ZROWS)],
                                out_hbm.at[c, pl.ds(off, _ZROWS)])

            return 0

        lax.fori_loop(0, _BLK_PER_TILE, wr, 0)

    return k(feat, w, den2, src, dst)


def _gat(x, src, dst, Wg, bg, attn_l, attn_r, n):
    feat = (x @ Wg).reshape(n, HEADS, H)
    el = (feat * attn_l[None, :, :]).sum(-1)
    er = (feat * attn_r[None, :, :]).sum(-1)
    e = jax.nn.leaky_relu(el[src] + er[dst], negative_slope=0.2)
    emax = jax.ops.segment_max(e, dst, num_segments=n)
    eexp = jnp.exp(e - emax[dst])
    denom = jax.ops.segment_sum(eexp, dst, num_segments=n)
    alpha = eexp / denom[dst]
    msg = feat[src] * alpha[:, :, None]
    out = jax.ops.segment_sum(msg, dst, num_segments=n)
    out = out + bg.reshape(1, HEADS, H)
    return out, alpha


_BN = 1000


def _norms_from(ds_ref):
    deg = ds_ref[0, :, 0:1] + ds_ref[1, :, 0:1]
    return jax.lax.rsqrt(jnp.maximum(deg, 1.0))


def _tck_in(x, W, dsrc):
    def body(x_ref, w_ref, ds_ref, o_ref):
        ns = _norms_from(ds_ref)
        o_ref[...] = jnp.dot(x_ref[...], w_ref[...],
                             preferred_element_type=jnp.float32) * ns

    return pl.pallas_call(
        body,
        grid=(N // _BN,),
        in_specs=[pl.BlockSpec((_BN, x.shape[1]), lambda i: (i, 0)),
                  pl.BlockSpec(W.shape, lambda i: (0, 0)),
                  pl.BlockSpec((2, _BN, _DW), lambda i: (0, i, 0))],
        out_specs=pl.BlockSpec((_BN, W.shape[1]), lambda i: (i, 0)),
        out_shape=jax.ShapeDtypeStruct((N, W.shape[1]), jnp.float32),
    )(x, W, dsrc)


def _tck_mid(aggp, ddst, dsrc, b2d, W):
    def body(a_ref, dd_ref, ds_ref, b_ref, w_ref, o_ref):
        nd = _norms_from(dd_ref)
        ns = _norms_from(ds_ref)
        hblk = jnp.maximum((a_ref[0] + a_ref[1]) * nd + b_ref[...], 0.0)
        o_ref[...] = jnp.dot(hblk, w_ref[...],
                             preferred_element_type=jnp.float32) * ns

    return pl.pallas_call(
        body,
        grid=(N // _BN,),
        in_specs=[pl.BlockSpec((2, _BN, H), lambda i: (0, i, 0)),
                  pl.BlockSpec((2, _BN, _DW), lambda i: (0, i, 0)),
                  pl.BlockSpec((2, _BN, _DW), lambda i: (0, i, 0)),
                  pl.BlockSpec((1, H), lambda i: (0, 0)),
                  pl.BlockSpec(W.shape, lambda i: (0, 0))],
        out_specs=pl.BlockSpec((_BN, W.shape[1]), lambda i: (i, 0)),
        out_shape=jax.ShapeDtypeStruct((N, W.shape[1]), jnp.float32),
    )(aggp, ddst, dsrc, b2d, W)


def _tck_gat_in(aggp, ddst, b2d, Wg, al2, ar2):
    def body(a_ref, dd_ref, b_ref, w_ref, al_ref, ar_ref, f_ref, ta_ref,
             tb_ref):
        nd = _norms_from(dd_ref)
        hblk = jnp.maximum((a_ref[0] + a_ref[1]) * nd + b_ref[...], 0.0)
        feat = jnp.dot(hblk, w_ref[...], preferred_element_type=jnp.float32)
        f_ref[...] = feat
        pl_ = feat * al_ref[...]
        pr_ = feat * ar_ref[...]
        ta_ref[...] = jnp.zeros_like(ta_ref)
        tb_ref[...] = jnp.zeros_like(tb_ref)
        for hh in range(HEADS):
            sl = slice(hh * H, (hh + 1) * H)
            ta_ref[:, hh:hh + 1] = jnp.sum(pl_[:, sl], axis=1, keepdims=True)
            tb_ref[:, hh:hh + 1] = jnp.sum(pr_[:, sl], axis=1, keepdims=True)

    return pl.pallas_call(
        body,
        grid=(N // _BN,),
        in_specs=[pl.BlockSpec((2, _BN, H), lambda i: (0, i, 0)),
                  pl.BlockSpec((2, _BN, _DW), lambda i: (0, i, 0)),
                  pl.BlockSpec((1, H), lambda i: (0, 0)),
                  pl.BlockSpec(Wg.shape, lambda i: (0, 0)),
                  pl.BlockSpec((1, HEADS * H), lambda i: (0, 0)),
                  pl.BlockSpec((1, HEADS * H), lambda i: (0, 0))],
        out_specs=[pl.BlockSpec((_BN, HEADS * H), lambda i: (i, 0)),
                   pl.BlockSpec((_BN, 16), lambda i: (i, 0)),
                   pl.BlockSpec((_BN, 16), lambda i: (i, 0))],
        out_shape=[jax.ShapeDtypeStruct((N, HEADS * H), jnp.float32),
                   jax.ShapeDtypeStruct((N, 16), jnp.float32),
                   jax.ShapeDtypeStruct((N, 16), jnp.float32)],
    )(aggp, ddst, b2d, Wg, al2, ar2)


def _tck_head(outp, bg2d, dsrc, W3):
    def body(o_ref, bg_ref, ds_ref, w3_ref, t_ref):
        ns = _norms_from(ds_ref)
        bgm = 0.25 * (bg_ref[:, 0:H] + bg_ref[:, H:2 * H]
                      + bg_ref[:, 2 * H:3 * H] + bg_ref[:, 3 * H:4 * H])
        hm = o_ref[0] + o_ref[1] + bgm
        t3 = jnp.dot(hm, w3_ref[...], preferred_element_type=jnp.float32) * ns
        t_ref[...] = jnp.zeros_like(t_ref)
        t_ref[:, 0:1] = t3

    return pl.pallas_call(
        body,
        grid=(N // _BN,),
        in_specs=[pl.BlockSpec((2, _BN, H), lambda i: (0, i, 0)),
                  pl.BlockSpec((1, HEADS * H), lambda i: (0, 0)),
                  pl.BlockSpec((2, _BN, _DW), lambda i: (0, i, 0)),
                  pl.BlockSpec((H, 1), lambda i: (0, 0))],
        out_specs=pl.BlockSpec((_BN, 16), lambda i: (i, 0)),
        out_shape=jax.ShapeDtypeStruct((N, 16), jnp.float32),
    )(outp, bg2d, dsrc, W3)


def _tck_final(agg3p, ddst, b3_2d):
    def body(a_ref, dd_ref, b3_ref, r_ref, rs_ref, cf_ref, ti_ref):
        degin = dd_ref[0, :, 0:1] + dd_ref[1, :, 0:1]
        nd = jax.lax.rsqrt(jnp.maximum(degin, 1.0))
        x = (a_ref[0, :, 0:1] + a_ref[1, :, 0:1]) * nd + b3_ref[0, 0]
        risk = 1.0 / (1.0 + jnp.exp(-x))
        r_ref[...] = risk
        rs_ref[...] = jnp.sum(risk, keepdims=True).reshape(1, 1) / float(N)
        nonempty = jnp.sum(jnp.where(degin > 0, 1.0, 0.0), keepdims=True)
        cf_ref[...] = nonempty.reshape(1, 1) / float(E)
        ii = jax.lax.broadcasted_iota(jnp.int32, (N, 1), 0)
        ii8 = jax.lax.broadcasted_iota(jnp.int32, (1, 8), 1)
        xv = risk
        acc = jnp.zeros((1, 8), jnp.int32)
        for kk in range(5):
            m = jnp.max(xv)
            ix = jnp.min(jnp.where(xv == m, ii, N))
            acc = jnp.where(ii8 == kk, ix, acc)
            xv = jnp.where(ii == ix, -1.0, xv)
        ti_ref[...] = acc

    return pl.pallas_call(
        body,
        in_specs=[pl.BlockSpec((2, N, _DW), lambda: (0, 0, 0)),
                  pl.BlockSpec((2, N, _DW), lambda: (0, 0, 0)),
                  pl.BlockSpec((1, 1), lambda: (0, 0))],
        out_specs=[pl.BlockSpec((N, 1), lambda: (0, 0)),
                   pl.BlockSpec((1, 1), lambda: (0, 0)),
                   pl.BlockSpec((1, 1), lambda: (0, 0)),
                   pl.BlockSpec((1, 8), lambda: (0, 0))],
        out_shape=[jax.ShapeDtypeStruct((N, 1), jnp.float32),
                   jax.ShapeDtypeStruct((1, 1), jnp.float32),
                   jax.ShapeDtypeStruct((1, 1), jnp.float32),
                   jax.ShapeDtypeStruct((1, 8), jnp.int32)],
    )(agg3p, ddst, b3_2d)


def kernel(features, edge_index, W1, b1, W2, b2, Wg, bg, attn_l, attn_r, W3, b3):
    src = edge_index[0]
    dst = edge_index[1]
    dsrc_p, ddst_p = _degrees(src, dst)

    t1 = _tck_in(features, W1, dsrc_p)
    agg1 = _seg_sum_rows(t1, src, dst, H)
    t2 = _tck_mid(agg1, ddst_p, dsrc_p, b1[None, :], W2)
    agg2 = _seg_sum_rows(t2, src, dst, H)
    feat2d, ta, tb = _tck_gat_in(agg2, ddst_p, b2[None, :], Wg,
                                 attn_l.reshape(1, HEADS * H),
                                 attn_r.reshape(1, HEADS * H))
    w, den_p = _gat_edge_weights(ta, tb, src, dst)
    out_p = _gat_aggregate(feat2d, w, den_p.reshape(2 * N, 16), src, dst)
    t3tab = _tck_head(out_p, bg[None, :], dsrc_p, W3)
    agg3 = _seg_sum_rows(t3tab, src, dst, 16)
    risk, rs, conf, topi = _tck_final(agg3, ddst_p, b3.reshape(1, 1))
    return risk, rs[0, 0], conf[0, 0], topi[0, :5]


# speedup vs baseline: 29.2501x; 1.1121x over previous
import functools

import jax
import jax.numpy as jnp
from jax import lax
from jax.experimental import pallas as pl
from jax.experimental.pallas import tpu as pltpu
from jax.experimental.pallas import tpu_sc as plsc

N = 10000
E = 320000
D = 128
H = 64
HEADS = 4

_NC = 2
_NS = 16
_C = 128
_E_SC = E // _NC
_CHUNKS = _E_SC // _C
_CH_PER_TILE = -(-_CHUNKS // _NS)
_ZROWS = 200
_NBLK = N // _ZROWS
_BLK_PER_TILE = -(-_NBLK // _NS)


def _zero_fill2d(buf, rows, width):
    z = jnp.zeros((16,), jnp.float32)
    per_row = width // 16

    def body(i, _):
        buf[i // per_row, pl.ds(pl.multiple_of((i % per_row) * 16, 16), 16)] = z
        return 0

    lax.fori_loop(0, rows * per_row, body, 0)


def _seg_sum_rows(table, src, dst, width):
    mesh = plsc.VectorSubcoreMesh(core_axis_name="c", subcore_axis_name="s")

    @functools.partial(
        pl.kernel,
        out_type=jax.ShapeDtypeStruct((_NC, N, width), jnp.float32),
        mesh=mesh,
        scratch_types=[
            pltpu.VMEM((_C,), jnp.int32),
            pltpu.VMEM((_C,), jnp.int32),
            pltpu.VMEM((_C,), jnp.int32),
            pltpu.VMEM((_C,), jnp.int32),
            pltpu.VMEM((_C, width), jnp.float32),
            pltpu.VMEM((_C, width), jnp.float32),
            pltpu.VMEM((_ZROWS, width), jnp.float32),
            pltpu.VMEM_SHARED((N, width), jnp.float32),
            pltpu.SemaphoreType.DMA,
            pltpu.SemaphoreType.DMA,
        ],
        compiler_params=pltpu.CompilerParams(use_tc_tiling_on_sc=False),
    )
    def k(table_hbm, src_hbm, dst_hbm, out_hbm, idx_s0, idx_s1, idx_d0,
          idx_d1, rows0, rows1, zbuf, acc_sh, sem0, sem1):
        idx_s = [idx_s0, idx_s1]
        idx_d = [idx_d0, idx_d1]
        rows = [rows0, rows1]
        sem = [sem0, sem1]
        c = lax.axis_index("c")
        s = lax.axis_index("s")

        _zero_fill2d(zbuf, _ZROWS, width)

        def zs(j, _):
            b = s + j * _NS

            @pl.when(b < _NBLK)
            def _():
                pltpu.sync_copy(zbuf, acc_sh.at[pl.ds(pl.multiple_of(b * _ZROWS, _ZROWS), _ZROWS)])

            return 0

        lax.fori_loop(0, _BLK_PER_TILE, zs, 0)
        plsc.subcore_barrier()

        def fetch(i, slot):
            kk = s + i * _NS

            @pl.when(kk < _CHUNKS)
            def _():
                base = c * _E_SC + kk * _C
                pltpu.sync_copy(src_hbm.at[pl.ds(base, _C)], idx_s[slot])
                pltpu.sync_copy(dst_hbm.at[pl.ds(base, _C)], idx_d[slot])
                pltpu.async_copy(table_hbm.at[idx_s[slot]], rows[slot],
                                 sem[slot])

        def consume(i, slot):
            kk = s + i * _NS

            @pl.when(kk < _CHUNKS)
            def _():
                pltpu.make_async_copy(table_hbm.at[idx_s[slot]], rows[slot],
                                      sem[slot]).wait()
                pltpu.sync_copy(rows[slot], acc_sh.at[idx_d[slot]], add=True)

        fetch(0, 0)

        def step2(i2, _):
            i = i2 * 2
            fetch(i + 1, 1)
            consume(i, 0)
            fetch(i + 2, 0)
            consume(i + 1, 1)
            return 0

        lax.fori_loop(0, (_CH_PER_TILE + 1) // 2, step2, 0)
        plsc.subcore_barrier()

        def wr(j, _):
            b = s + j * _NS

            @pl.when(b < _NBLK)
            def _():
                off = pl.multiple_of(b * _ZROWS, _ZROWS)
                pltpu.sync_copy(acc_sh.at[pl.ds(off, _ZROWS)],
                                out_hbm.at[c, pl.ds(off, _ZROWS)])

            return 0

        lax.fori_loop(0, _BLK_PER_TILE, wr, 0)

    return k(table, src, dst)


_DW = 16


def _degrees(src, dst):
    mesh = plsc.VectorSubcoreMesh(core_axis_name="c", subcore_axis_name="s")

    @functools.partial(
        pl.kernel,
        out_type=(jax.ShapeDtypeStruct((_NC, N, _DW), jnp.float32),
                  jax.ShapeDtypeStruct((_NC, N, _DW), jnp.float32)),
        mesh=mesh,
        scratch_types=[
            pltpu.VMEM((_C,), jnp.int32),
            pltpu.VMEM((_C,), jnp.int32),
            pltpu.VMEM((_C, _DW), jnp.float32),
            pltpu.VMEM((_ZROWS, _DW), jnp.float32),
            pltpu.VMEM_SHARED((N, _DW), jnp.float32),
            pltpu.VMEM_SHARED((N, _DW), jnp.float32),
        ],
        compiler_params=pltpu.CompilerParams(use_tc_tiling_on_sc=False),
    )
    def k(src_hbm, dst_hbm, osrc_hbm, odst_hbm, idx_s, idx_d, ones, zbuf,
          acc_s, acc_d):
        c = lax.axis_index("c")
        s = lax.axis_index("s")

        _zero_fill2d(zbuf, _ZROWS, _DW)
        o = jnp.ones((16,), jnp.float32)

        def fill(i, _):
            ones[i, pl.ds(0, 16)] = o
            return 0

        lax.fori_loop(0, _C, fill, 0)

        def zs(j, _):
            b = s + j * _NS

            @pl.when(b < _NBLK)
            def _():
                off = pl.multiple_of(b * _ZROWS, _ZROWS)
                pltpu.sync_copy(zbuf, acc_s.at[pl.ds(off, _ZROWS)])
                pltpu.sync_copy(zbuf, acc_d.at[pl.ds(off, _ZROWS)])

            return 0

        lax.fori_loop(0, _BLK_PER_TILE, zs, 0)
        plsc.subcore_barrier()

        def chunk(i, _):
            kk = s + i * _NS

            @pl.when(kk < _CHUNKS)
            def _():
                base = c * _E_SC + kk * _C
                pltpu.sync_copy(src_hbm.at[pl.ds(base, _C)], idx_s)
                pltpu.sync_copy(dst_hbm.at[pl.ds(base, _C)], idx_d)
                pltpu.sync_copy(ones, acc_s.at[idx_s], add=True)
                pltpu.sync_copy(ones, acc_d.at[idx_d], add=True)

            return 0

        lax.fori_loop(0, _CH_PER_TILE, chunk, 0)
        plsc.subcore_barrier()

        def wr(j, _):
            b = s + j * _NS

            @pl.when(b < _NBLK)
            def _():
                off = pl.multiple_of(b * _ZROWS, _ZROWS)
                pltpu.sync_copy(acc_s.at[pl.ds(off, _ZROWS)],
                                osrc_hbm.at[c, pl.ds(off, _ZROWS)])
                pltpu.sync_copy(acc_d.at[pl.ds(off, _ZROWS)],
                                odst_hbm.at[c, pl.ds(off, _ZROWS)])

            return 0

        lax.fori_loop(0, _BLK_PER_TILE, wr, 0)

    return k(src, dst)


def _gc(x, src, dst, W, b, n, norm_src, norm_dst, width):
    h = x @ W
    if width > h.shape[1]:
        h = jnp.broadcast_to(h, (n, width))
    table = h * norm_src[:, None]
    parts = _seg_sum_rows(table, src, dst, width)
    agg = parts[0] + parts[1]
    return agg * norm_dst[:, None] + b


def _gat_edge_weights(ta, tb, src, dst):
    mesh = plsc.VectorSubcoreMesh(core_axis_name="c", subcore_axis_name="s")

    @functools.partial(
        pl.kernel,
        out_type=(jax.ShapeDtypeStruct((E, 16), jnp.float32),
                  jax.ShapeDtypeStruct((_NC, N, 16), jnp.float32)),
        mesh=mesh,
        scratch_types=[
            pltpu.VMEM((_C,), jnp.int32),
            pltpu.VMEM((_C,), jnp.int32),
            pltpu.VMEM((_C, 16), jnp.float32),
            pltpu.VMEM((_C, 16), jnp.float32),
            pltpu.VMEM((_C, 16), jnp.float32),
            pltpu.VMEM((_ZROWS, 16), jnp.float32),
            pltpu.VMEM_SHARED((N, 16), jnp.float32),
            pltpu.SemaphoreType.DMA,
        ],
        compiler_params=pltpu.CompilerParams(use_tc_tiling_on_sc=False),
    )
    def k(ta_hbm, tb_hbm, src_hbm, dst_hbm, w_hbm, den_hbm, idx_s, idx_d,
          es, ed, wbuf, zbuf, acc_sh, sem):
        c = lax.axis_index("c")
        s = lax.axis_index("s")

        _zero_fill2d(zbuf, _ZROWS, 16)

        def zs(j, _):
            b = s + j * _NS

            @pl.when(b < _NBLK)
            def _():
                off = pl.multiple_of(b * _ZROWS, _ZROWS)
                pltpu.sync_copy(zbuf, acc_sh.at[pl.ds(off, _ZROWS)])

            return 0

        lax.fori_loop(0, _BLK_PER_TILE, zs, 0)
        plsc.subcore_barrier()

        def chunk(i, _):
            kk = s + i * _NS

            @pl.when(kk < _CHUNKS)
            def _():
                base = c * _E_SC + kk * _C
                pltpu.sync_copy(src_hbm.at[pl.ds(base, _C)], idx_s)
                pltpu.sync_copy(dst_hbm.at[pl.ds(base, _C)], idx_d)
                cp1 = pltpu.async_copy(ta_hbm.at[idx_s], es, sem)
                cp2 = pltpu.async_copy(tb_hbm.at[idx_d], ed, sem)
                cp1.wait()
                cp2.wait()

                def pe(j, _):
                    e2 = es[j, pl.ds(0, 16)] + ed[j, pl.ds(0, 16)]
                    e2 = jnp.where(e2 > 0, e2, 0.2 * e2)
                    wbuf[j, pl.ds(0, 16)] = jnp.exp(e2)
                    return 0

                lax.fori_loop(0, _C, pe, 0)
                pltpu.sync_copy(wbuf, w_hbm.at[pl.ds(base, _C)])
                pltpu.sync_copy(wbuf, acc_sh.at[idx_d], add=True)

            return 0

        lax.fori_loop(0, _CH_PER_TILE, chunk, 0)
        plsc.subcore_barrier()

        def wr(j, _):
            b = s + j * _NS

            @pl.when(b < _NBLK)
            def _():
                off = pl.multiple_of(b * _ZROWS, _ZROWS)
                pltpu.sync_copy(acc_sh.at[pl.ds(off, _ZROWS)],
                                den_hbm.at[c, pl.ds(off, _ZROWS)])

            return 0

        lax.fori_loop(0, _BLK_PER_TILE, wr, 0)

    return k(ta, tb, src, dst)


def _gat_aggregate(feat, w, den2, src, dst):
    mesh = plsc.VectorSubcoreMesh(core_axis_name="c", subcore_axis_name="s")

    @functools.partial(
        pl.kernel,
        out_type=jax.ShapeDtypeStruct((_NC, N, H), jnp.float32),
        mesh=mesh,
        scratch_types=[
            pltpu.VMEM((_C,), jnp.int32),
            pltpu.VMEM((_C,), jnp.int32),
            pltpu.VMEM((_C,), jnp.int32),
            pltpu.VMEM((_C, HEADS * H), jnp.float32),
            pltpu.VMEM((_C, 16), jnp.float32),
            pltpu.VMEM((_C, 16), jnp.float32),
            pltpu.VMEM((_C, 16), jnp.float32),
            pltpu.VMEM((_C, H), jnp.float32),
            pltpu.VMEM((_ZROWS, H), jnp.float32),
            pltpu.VMEM_SHARED((N, H), jnp.float32),
            pltpu.SemaphoreType.DMA,
        ],
        compiler_params=pltpu.CompilerParams(use_tc_tiling_on_sc=False),
    )
    def k(feat_hbm, w_hbm, den_hbm, src_hbm, dst_hbm, out_hbm, idx_s, idx_d,
          idx_d2, frows, wch, d0, d1, msg, zbuf, acc_sh, sem):
        c = lax.axis_index("c")
        s = lax.axis_index("s")

        _zero_fill2d(zbuf, _ZROWS, H)

        def zs(j, _):
            b = s + j * _NS

            @pl.when(b < _NBLK)
            def _():
                off = pl.multiple_of(b * _ZROWS, _ZROWS)
                pltpu.sync_copy(zbuf, acc_sh.at[pl.ds(off, _ZROWS)])

            return 0

        lax.fori_loop(0, _BLK_PER_TILE, zs, 0)
        plsc.subcore_barrier()

        bidx = [jnp.full((16,), h, jnp.int32) for h in range(HEADS)]

        def chunk(i, _):
            kk = s + i * _NS

            @pl.when(kk < _CHUNKS)
            def _():
                base = c * _E_SC + kk * _C
                pltpu.sync_copy(src_hbm.at[pl.ds(base, _C)], idx_s)
                pltpu.sync_copy(dst_hbm.at[pl.ds(base, _C)], idx_d)

                def sh(q, _):
                    off = pl.multiple_of(q * 16, 16)
                    idx_d2[pl.ds(off, 16)] = idx_d[pl.ds(off, 16)] + N
                    return 0

                lax.fori_loop(0, _C // 16, sh, 0)

                cp1 = pltpu.async_copy(feat_hbm.at[idx_s], frows, sem)
                pltpu.sync_copy(w_hbm.at[pl.ds(base, _C)], wch)
                pltpu.sync_copy(den_hbm.at[idx_d], d0)
                pltpu.sync_copy(den_hbm.at[idx_d2], d1)
                cp1.wait()

                def pe(j, _):
                    dv = d0[j, pl.ds(0, 16)] + d1[j, pl.ds(0, 16)]
                    dv = jnp.where(dv == 0.0, 1.0, dv)
                    cf = 0.25 * wch[j, pl.ds(0, 16)] / dv
                    cb = [cf.at[bidx[h]].get(
                        mode=jax.lax.GatherScatterMode.PROMISE_IN_BOUNDS)
                        for h in range(HEADS)]
                    for q in range(H // 16):
                        o = q * 16
                        a = cb[0] * frows[j, pl.ds(0 * H + o, 16)]
                        a = a + cb[1] * frows[j, pl.ds(1 * H + o, 16)]
                        a = a + cb[2] * frows[j, pl.ds(2 * H + o, 16)]
                        a = a + cb[3] * frows[j, pl.ds(3 * H + o, 16)]
                        msg[j, pl.ds(o, 16)] = a
                    return 0

                lax.fori_loop(0, _C, pe, 0)
                pltpu.sync_copy(msg, acc_sh.at[idx_d], add=True)

            return 0

        lax.fori_loop(0, _CH_PER_TILE, chunk, 0)
        plsc.subcore_barrier()

        def wr(j, _):
            b = s + j * _NS

            @pl.when(b < _NBLK)
            def _():
                off = pl.multiple_of(b * _ZROWS, _ZROWS)
                pltpu.sync_copy(acc_sh.at[pl.ds(off, _ZROWS)],
                                out_hbm.at[c, pl.ds(off, _ZROWS)])

            return 0

        lax.fori_loop(0, _BLK_PER_TILE, wr, 0)

    return k(feat, w, den2, src, dst)


_BN = 1000


def _norms_from(ds_ref):
    deg = ds_ref[0, :, 0:1] + ds_ref[1, :, 0:1]
    return jax.lax.rsqrt(jnp.maximum(deg, 1.0))


def _tck_in(x, W, dsrc):
    def body(x_ref, w_ref, ds_ref, o_ref):
        ns = _norms_from(ds_ref)
        o_ref[...] = jnp.dot(x_ref[...], w_ref[...],
                             preferred_element_type=jnp.float32) * ns

    return pl.pallas_call(
        body,
        grid=(N // _BN,),
        in_specs=[pl.BlockSpec((_BN, x.shape[1]), lambda i: (i, 0)),
                  pl.BlockSpec(W.shape, lambda i: (0, 0)),
                  pl.BlockSpec((2, _BN, _DW), lambda i: (0, i, 0))],
        out_specs=pl.BlockSpec((_BN, W.shape[1]), lambda i: (i, 0)),
        out_shape=jax.ShapeDtypeStruct((N, W.shape[1]), jnp.float32),
    )(x, W, dsrc)


def _tck_mid(aggp, ddst, dsrc, b2d, W):
    def body(a_ref, dd_ref, ds_ref, b_ref, w_ref, o_ref):
        nd = _norms_from(dd_ref)
        ns = _norms_from(ds_ref)
        hblk = jnp.maximum((a_ref[0] + a_ref[1]) * nd + b_ref[...], 0.0)
        o_ref[...] = jnp.dot(hblk, w_ref[...],
                             preferred_element_type=jnp.float32) * ns

    return pl.pallas_call(
        body,
        grid=(N // _BN,),
        in_specs=[pl.BlockSpec((2, _BN, H), lambda i: (0, i, 0)),
                  pl.BlockSpec((2, _BN, _DW), lambda i: (0, i, 0)),
                  pl.BlockSpec((2, _BN, _DW), lambda i: (0, i, 0)),
                  pl.BlockSpec((1, H), lambda i: (0, 0)),
                  pl.BlockSpec(W.shape, lambda i: (0, 0))],
        out_specs=pl.BlockSpec((_BN, W.shape[1]), lambda i: (i, 0)),
        out_shape=jax.ShapeDtypeStruct((N, W.shape[1]), jnp.float32),
    )(aggp, ddst, dsrc, b2d, W)


def _tck_gat_in(aggp, ddst, b2d, Wg, al2, ar2):
    def body(a_ref, dd_ref, b_ref, w_ref, al_ref, ar_ref, f_ref, ta_ref,
             tb_ref):
        nd = _norms_from(dd_ref)
        hblk = jnp.maximum((a_ref[0] + a_ref[1]) * nd + b_ref[...], 0.0)
        feat = jnp.dot(hblk, w_ref[...], preferred_element_type=jnp.float32)
        f_ref[...] = feat
        pl_ = feat * al_ref[...]
        pr_ = feat * ar_ref[...]
        ta_ref[...] = jnp.zeros_like(ta_ref)
        tb_ref[...] = jnp.zeros_like(tb_ref)
        for hh in range(HEADS):
            sl = slice(hh * H, (hh + 1) * H)
            ta_ref[:, hh:hh + 1] = jnp.sum(pl_[:, sl], axis=1, keepdims=True)
            tb_ref[:, hh:hh + 1] = jnp.sum(pr_[:, sl], axis=1, keepdims=True)

    return pl.pallas_call(
        body,
        grid=(N // _BN,),
        in_specs=[pl.BlockSpec((2, _BN, H), lambda i: (0, i, 0)),
                  pl.BlockSpec((2, _BN, _DW), lambda i: (0, i, 0)),
                  pl.BlockSpec((1, H), lambda i: (0, 0)),
                  pl.BlockSpec(Wg.shape, lambda i: (0, 0)),
                  pl.BlockSpec((1, HEADS * H), lambda i: (0, 0)),
                  pl.BlockSpec((1, HEADS * H), lambda i: (0, 0))],
        out_specs=[pl.BlockSpec((_BN, HEADS * H), lambda i: (i, 0)),
                   pl.BlockSpec((_BN, 16), lambda i: (i, 0)),
                   pl.BlockSpec((_BN, 16), lambda i: (i, 0))],
        out_shape=[jax.ShapeDtypeStruct((N, HEADS * H), jnp.float32),
                   jax.ShapeDtypeStruct((N, 16), jnp.float32),
                   jax.ShapeDtypeStruct((N, 16), jnp.float32)],
    )(aggp, ddst, b2d, Wg, al2, ar2)


def _tck_head(outp, bg2d, dsrc, W3):
    def body(o_ref, bg_ref, ds_ref, w3_ref, t_ref):
        ns = _norms_from(ds_ref)
        bgm = 0.25 * (bg_ref[:, 0:H] + bg_ref[:, H:2 * H]
                      + bg_ref[:, 2 * H:3 * H] + bg_ref[:, 3 * H:4 * H])
        hm = o_ref[0] + o_ref[1] + bgm
        t3 = jnp.dot(hm, w3_ref[...], preferred_element_type=jnp.float32) * ns
        t_ref[...] = jnp.zeros_like(t_ref)
        t_ref[:, 0:1] = t3

    return pl.pallas_call(
        body,
        grid=(N // _BN,),
        in_specs=[pl.BlockSpec((2, _BN, H), lambda i: (0, i, 0)),
                  pl.BlockSpec((1, HEADS * H), lambda i: (0, 0)),
                  pl.BlockSpec((2, _BN, _DW), lambda i: (0, i, 0)),
                  pl.BlockSpec((H, 1), lambda i: (0, 0))],
        out_specs=pl.BlockSpec((_BN, 16), lambda i: (i, 0)),
        out_shape=jax.ShapeDtypeStruct((N, 16), jnp.float32),
    )(outp, bg2d, dsrc, W3)


def _tck_final(agg3p, ddst, b3_2d):
    def body(a_ref, dd_ref, b3_ref, r_ref, rs_ref, cf_ref, ti_ref):
        degin = dd_ref[0, :, 0:1] + dd_ref[1, :, 0:1]
        nd = jax.lax.rsqrt(jnp.maximum(degin, 1.0))
        x = (a_ref[0, :, 0:1] + a_ref[1, :, 0:1]) * nd + b3_ref[0, 0]
        risk = 1.0 / (1.0 + jnp.exp(-x))
        r_ref[...] = risk
        rs_ref[...] = jnp.sum(risk, keepdims=True).reshape(1, 1) / float(N)
        nonempty = jnp.sum(jnp.where(degin > 0, 1.0, 0.0), keepdims=True)
        cf_ref[...] = nonempty.reshape(1, 1) / float(E)
        ii = jax.lax.broadcasted_iota(jnp.int32, (N, 1), 0)
        ii8 = jax.lax.broadcasted_iota(jnp.int32, (1, 8), 1)
        xv = risk
        acc = jnp.zeros((1, 8), jnp.int32)
        for kk in range(5):
            m = jnp.max(xv)
            ix = jnp.min(jnp.where(xv == m, ii, N))
            acc = jnp.where(ii8 == kk, ix, acc)
            xv = jnp.where(ii == ix, -1.0, xv)
        ti_ref[...] = acc

    return pl.pallas_call(
        body,
        in_specs=[pl.BlockSpec((2, N, _DW), lambda: (0, 0, 0)),
                  pl.BlockSpec((2, N, _DW), lambda: (0, 0, 0)),
                  pl.BlockSpec((1, 1), lambda: (0, 0))],
        out_specs=[pl.BlockSpec((N, 1), lambda: (0, 0)),
                   pl.BlockSpec((1, 1), lambda: (0, 0)),
                   pl.BlockSpec((1, 1), lambda: (0, 0)),
                   pl.BlockSpec((1, 8), lambda: (0, 0))],
        out_shape=[jax.ShapeDtypeStruct((N, 1), jnp.float32),
                   jax.ShapeDtypeStruct((1, 1), jnp.float32),
                   jax.ShapeDtypeStruct((1, 1), jnp.float32),
                   jax.ShapeDtypeStruct((1, 8), jnp.int32)],
    )(agg3p, ddst, b3_2d)


def kernel(features, edge_index, W1, b1, W2, b2, Wg, bg, attn_l, attn_r, W3, b3):
    src = edge_index[0]
    dst = edge_index[1]
    dsrc_p, ddst_p = _degrees(src, dst)

    t1 = _tck_in(features, W1, dsrc_p)
    agg1 = _seg_sum_rows(t1, src, dst, H)
    t2 = _tck_mid(agg1, ddst_p, dsrc_p, b1[None, :], W2)
    agg2 = _seg_sum_rows(t2, src, dst, H)
    feat2d, ta, tb = _tck_gat_in(agg2, ddst_p, b2[None, :], Wg,
                                 attn_l.reshape(1, HEADS * H),
                                 attn_r.reshape(1, HEADS * H))
    w, den_p = _gat_edge_weights(ta, tb, src, dst)
    out_p = _gat_aggregate(feat2d, w, den_p.reshape(2 * N, 16), src, dst)
    t3tab = _tck_head(out_p, bg[None, :], dsrc_p, W3)
    agg3 = _seg_sum_rows(t3tab, src, dst, 16)
    risk, rs, conf, topi = _tck_final(agg3, ddst_p, b3.reshape(1, 1))
    return risk, rs[0, 0], conf[0, 0], topi[0, :5]
